# baseline (device time: 53723 ns/iter reference)
import jax
import jax.numpy as jnp
from jax import lax
from jax.experimental import pallas as pl
from jax.experimental.pallas import tpu as pltpu

PART = 1024
CHUNKS = (128, 224, 336, 336)
NCH = len(CHUNKS)
A_LEN = 352
B_OFF, B_LEN = 352, 336
C_OFF, C_LEN = 688, 336
NSEM = NCH + 1
NLOAD = 5
NSTORE = 7


def kernel(x):
    m, n = x.shape

    def body(
        x_ref, out_ref, vbuf, xv, load_sem, store_sem,
        x_send, x_recv, y_send, y_recv, z_send, z_recv,
    ):
        my_x = lax.axis_index("x")
        my_y = lax.axis_index("y")
        my_z = lax.axis_index("z")
        zbit = my_z % 2
        zp = my_z + 1 - 2 * zbit
        x_peer = (1 - my_x, my_y, my_z)
        y_peer = (my_x, 1 - my_y, my_z)
        z_peer = (my_x, my_y, zp)

        l0 = my_y + 2 * zbit
        l1 = (1 - my_y) + 2 * zbit
        l2 = my_y + 2 * (1 - zbit)
        l3 = (1 - my_y) + 2 * (1 - zbit)

        own = my_x * m
        fgn = (1 - my_x) * m

        load_specs = (
            (l0 * PART, PART),
            (l3 * PART, A_LEN),
            (l1 * PART, PART),
            (l2 * PART, PART),
            (l3 * PART + A_LEN, PART - A_LEN),
        )
        loads = []
        for i, (base, ln) in enumerate(load_specs):
            cp = pltpu.make_async_copy(
                x_ref.at[pl.ds(base, ln)],
                xv.at[pl.ds(base, ln)],
                load_sem.at[i],
            )
            cp.start()
            loads.append(cp)

        barrier_sem = pltpu.get_barrier_semaphore()
        for nbr in (x_peer, y_peer, z_peer):
            pl.semaphore_signal(
                barrier_sem, inc=1, device_id=nbr,
                device_id_type=pl.DeviceIdType.MESH,
            )
        pl.semaphore_wait(barrier_sem, 3)

        def cast(base, length):
            vbuf[pl.ds(own + base, length), :] = (
                xv[pl.ds(base, length), :].astype(vbuf.dtype)
            )

        def rdma(rows, sends, recvs, k, dev):
            return pltpu.make_async_remote_copy(
                src_ref=vbuf.at[rows],
                dst_ref=vbuf.at[rows],
                send_sem=sends.at[k],
                recv_sem=recvs.at[k],
                device_id=dev,
                device_id_type=pl.DeviceIdType.MESH,
            )

        stores = []

        def store(rows):
            cp = pltpu.make_async_copy(
                vbuf.at[rows], out_ref.at[rows], store_sem.at[len(stores)]
            )
            cp.start()
            stores.append(cp)

        loads[0].wait()
        xs = []
        off = 0
        for k, sz in enumerate(CHUNKS):
            cast(l0 * PART + off, sz)
            r = rdma(
                pl.ds(own + l0 * PART + off, sz), x_send, x_recv, k, x_peer
            )
            r.start()
            xs.append(r)
            off += sz
        loads[1].wait()
        cast(l3 * PART, A_LEN)
        r = rdma(pl.ds(own + l3 * PART, A_LEN), x_send, x_recv, NCH, x_peer)
        r.start()
        xs.append(r)
        loads[2].wait()
        cast(l1 * PART, PART)
        loads[3].wait()
        cast(l2 * PART, PART)
        loads[4].wait()
        cast(l3 * PART + A_LEN, PART - A_LEN)
        store(pl.ds(own, m))

        ys, zs = [], []
        off = 0
        for k, sz in enumerate(CHUNKS):
            xs[k].wait_recv()
            rows = pl.ds(fgn + l0 * PART + off, sz)
            ry = rdma(rows, y_send, y_recv, k, y_peer)
            ry.start()
            ys.append(ry)
            rz = rdma(rows, z_send, z_recv, k, z_peer)
            rz.start()
            zs.append(rz)
            off += sz
        store(pl.ds(fgn + l0 * PART, PART))

        zs[2].wait_recv()
        ry = rdma(
            pl.ds(fgn + l2 * PART + B_OFF, B_LEN), y_send, y_recv, NCH, y_peer
        )
        ry.start()
        ys.append(ry)
        for k in (0, 1, 2, 3):
            ys[k].wait_recv()
        rz = rdma(
            pl.ds(fgn + l1 * PART + C_OFF, C_LEN), z_send, z_recv, NCH, z_peer
        )
        rz.start()
        zs.append(rz)
        store(pl.ds(fgn + l1 * PART, PART))

        xs[NCH].wait_recv()
        store(pl.ds(fgn + l3 * PART, A_LEN))
        for k in (0, 1, 3):
            zs[k].wait_recv()
        store(pl.ds(fgn + l2 * PART, PART))
        ys[NCH].wait_recv()
        store(pl.ds(fgn + l3 * PART + B_OFF, B_LEN))
        zs[NCH].wait_recv()
        store(pl.ds(fgn + l3 * PART + C_OFF, C_LEN))
        for k in range(NSEM):
            xs[k].wait_send()
            ys[k].wait_send()
            zs[k].wait_send()
        for cp in stores:
            cp.wait()

    return pl.pallas_call(
        body,
        out_shape=jax.ShapeDtypeStruct((2 * m, n), jnp.bfloat16),
        in_specs=[pl.BlockSpec(memory_space=pl.ANY)],
        out_specs=pl.BlockSpec(memory_space=pl.ANY),
        scratch_shapes=[
            pltpu.VMEM((2 * m, n), jnp.bfloat16),
            pltpu.VMEM((m, n), x.dtype),
            pltpu.SemaphoreType.DMA((NLOAD,)),
            pltpu.SemaphoreType.DMA((NSTORE,)),
            pltpu.SemaphoreType.DMA((NSEM,)),
            pltpu.SemaphoreType.DMA((NSEM,)),
            pltpu.SemaphoreType.DMA((NSEM,)),
            pltpu.SemaphoreType.DMA((NSEM,)),
            pltpu.SemaphoreType.DMA((NSEM,)),
            pltpu.SemaphoreType.DMA((NSEM,)),
        ],
        compiler_params=pltpu.CompilerParams(collective_id=0),
    )(x)


# device time: 53416 ns/iter; 1.0057x vs baseline; 1.0057x over previous
import jax
import jax.numpy as jnp
from jax import lax
from jax.experimental import pallas as pl
from jax.experimental.pallas import tpu as pltpu

PART = 1024
CHUNKS = (128, 224, 336, 336)
NCH = len(CHUNKS)
A_LEN = 352
B_OFF, B_LEN = 352, 336
C_OFF, C_LEN = 688, 336
NSEM = NCH + 1
NLOAD = 5


def kernel(x):
    m, n = x.shape

    def body(
        x_ref, out_ref, xv, load_sem,
        x_send, x_recv, y_send, y_recv, z_send, z_recv,
    ):
        my_x = lax.axis_index("x")
        my_y = lax.axis_index("y")
        my_z = lax.axis_index("z")
        zbit = my_z % 2
        zp = my_z + 1 - 2 * zbit
        x_peer = (1 - my_x, my_y, my_z)
        y_peer = (my_x, 1 - my_y, my_z)
        z_peer = (my_x, my_y, zp)

        l0 = my_y + 2 * zbit
        l1 = (1 - my_y) + 2 * zbit
        l2 = my_y + 2 * (1 - zbit)
        l3 = (1 - my_y) + 2 * (1 - zbit)

        own = my_x * m
        fgn = (1 - my_x) * m

        load_specs = (
            (l0 * PART, PART),
            (l3 * PART, A_LEN),
            (l1 * PART, PART),
            (l2 * PART, PART),
            (l3 * PART + A_LEN, PART - A_LEN),
        )
        loads = []
        for i, (base, ln) in enumerate(load_specs):
            cp = pltpu.make_async_copy(
                x_ref.at[pl.ds(base, ln)],
                xv.at[pl.ds(base, ln)],
                load_sem.at[i],
            )
            cp.start()
            loads.append(cp)

        barrier_sem = pltpu.get_barrier_semaphore()
        for nbr in (x_peer, y_peer, z_peer):
            pl.semaphore_signal(
                barrier_sem, inc=1, device_id=nbr,
                device_id_type=pl.DeviceIdType.MESH,
            )
        pl.semaphore_wait(barrier_sem, 3)

        def cast(base, length):
            out_ref[pl.ds(own + base, length), :] = (
                xv[pl.ds(base, length), :].astype(out_ref.dtype)
            )

        def rdma(rows, sends, recvs, k, dev):
            return pltpu.make_async_remote_copy(
                src_ref=out_ref.at[rows],
                dst_ref=out_ref.at[rows],
                send_sem=sends.at[k],
                recv_sem=recvs.at[k],
                device_id=dev,
                device_id_type=pl.DeviceIdType.MESH,
            )

        loads[0].wait()
        xs = []
        off = 0
        for k, sz in enumerate(CHUNKS):
            cast(l0 * PART + off, sz)
            r = rdma(
                pl.ds(own + l0 * PART + off, sz), x_send, x_recv, k, x_peer
            )
            r.start()
            xs.append(r)
            off += sz
        loads[1].wait()
        cast(l3 * PART, A_LEN)
        r = rdma(pl.ds(own + l3 * PART, A_LEN), x_send, x_recv, NCH, x_peer)
        r.start()
        xs.append(r)
        loads[2].wait()
        cast(l1 * PART, PART)
        loads[3].wait()
        cast(l2 * PART, PART)
        loads[4].wait()
        cast(l3 * PART + A_LEN, PART - A_LEN)

        ys, zs = [], []
        off = 0
        for k, sz in enumerate(CHUNKS):
            xs[k].wait_recv()
            rows = pl.ds(fgn + l0 * PART + off, sz)
            ry = rdma(rows, y_send, y_recv, k, y_peer)
            ry.start()
            ys.append(ry)
            rz = rdma(rows, z_send, z_recv, k, z_peer)
            rz.start()
            zs.append(rz)
            off += sz

        zs[2].wait_recv()
        ry = rdma(
            pl.ds(fgn + l2 * PART + B_OFF, B_LEN), y_send, y_recv, NCH, y_peer
        )
        ry.start()
        ys.append(ry)
        ys[3].wait_recv()
        rz = rdma(
            pl.ds(fgn + l1 * PART + C_OFF, C_LEN), z_send, z_recv, NCH, z_peer
        )
        rz.start()
        zs.append(rz)

        xs[NCH].wait_recv()
        for k in (0, 1, 2, NCH):
            ys[k].wait_recv()
        for k in (0, 1, 3, NCH):
            zs[k].wait_recv()
        for k in range(NSEM):
            xs[k].wait_send()
            ys[k].wait_send()
            zs[k].wait_send()

    return pl.pallas_call(
        body,
        out_shape=jax.ShapeDtypeStruct((2 * m, n), jnp.bfloat16),
        in_specs=[pl.BlockSpec(memory_space=pl.ANY)],
        out_specs=pl.BlockSpec(memory_space=pltpu.VMEM),
        scratch_shapes=[
            pltpu.VMEM((m, n), x.dtype),
            pltpu.SemaphoreType.DMA((NLOAD,)),
            pltpu.SemaphoreType.DMA((NSEM,)),
            pltpu.SemaphoreType.DMA((NSEM,)),
            pltpu.SemaphoreType.DMA((NSEM,)),
            pltpu.SemaphoreType.DMA((NSEM,)),
            pltpu.SemaphoreType.DMA((NSEM,)),
            pltpu.SemaphoreType.DMA((NSEM,)),
        ],
        compiler_params=pltpu.CompilerParams(collective_id=0),
    )(x)


# device time: 50539 ns/iter; 1.0630x vs baseline; 1.0569x over previous
import jax
import jax.numpy as jnp
from jax import lax
from jax.experimental import pallas as pl
from jax.experimental.pallas import tpu as pltpu

PART = 1024
CHUNKS = (256, 256, 320, 192)
NCH = len(CHUNKS)
A_LEN = 512
B_OFF, B_LEN = 512, 320
C_OFF, C_LEN = 832, 192
NSEM = NCH + 1
NLOAD = 5


def kernel(x):
    m, n = x.shape

    def body(
        x_ref, out_ref, xv, load_sem,
        x_send, x_recv, y_send, y_recv, z_send, z_recv,
    ):
        my_x = lax.axis_index("x")
        my_y = lax.axis_index("y")
        my_z = lax.axis_index("z")
        zbit = my_z % 2
        zp = my_z + 1 - 2 * zbit
        x_peer = (1 - my_x, my_y, my_z)
        y_peer = (my_x, 1 - my_y, my_z)
        z_peer = (my_x, my_y, zp)

        l0 = my_y + 2 * zbit
        l1 = (1 - my_y) + 2 * zbit
        l2 = my_y + 2 * (1 - zbit)
        l3 = (1 - my_y) + 2 * (1 - zbit)

        own = my_x * m
        fgn = (1 - my_x) * m

        load_specs = (
            (l0 * PART, PART),
            (l3 * PART, A_LEN),
            (l1 * PART, PART),
            (l2 * PART, PART),
            (l3 * PART + A_LEN, PART - A_LEN),
        )
        loads = []
        for i, (base, ln) in enumerate(load_specs):
            cp = pltpu.make_async_copy(
                x_ref.at[pl.ds(base, ln)],
                xv.at[pl.ds(base, ln)],
                load_sem.at[i],
            )
            cp.start()
            loads.append(cp)

        barrier_sem = pltpu.get_barrier_semaphore()
        for nbr in (x_peer, y_peer, z_peer):
            pl.semaphore_signal(
                barrier_sem, inc=1, device_id=nbr,
                device_id_type=pl.DeviceIdType.MESH,
            )
        pl.semaphore_wait(barrier_sem, 3)

        def cast(base, length):
            out_ref[pl.ds(own + base, length), :] = (
                xv[pl.ds(base, length), :].astype(out_ref.dtype)
            )

        def rdma(rows, sends, recvs, k, dev):
            return pltpu.make_async_remote_copy(
                src_ref=out_ref.at[rows],
                dst_ref=out_ref.at[rows],
                send_sem=sends.at[k],
                recv_sem=recvs.at[k],
                device_id=dev,
                device_id_type=pl.DeviceIdType.MESH,
            )

        loads[0].wait()
        xs = []
        off = 0
        for k, sz in enumerate(CHUNKS):
            cast(l0 * PART + off, sz)
            r = rdma(
                pl.ds(own + l0 * PART + off, sz), x_send, x_recv, k, x_peer
            )
            r.start()
            xs.append(r)
            off += sz
        loads[1].wait()
        cast(l3 * PART, A_LEN)
        r = rdma(pl.ds(own + l3 * PART, A_LEN), x_send, x_recv, NCH, x_peer)
        r.start()
        xs.append(r)
        loads[2].wait()
        cast(l1 * PART, PART)
        loads[3].wait()
        cast(l2 * PART, PART)
        loads[4].wait()
        cast(l3 * PART + A_LEN, PART - A_LEN)

        ys, zs = [], []
        off = 0
        for k, sz in enumerate(CHUNKS):
            xs[k].wait_recv()
            rows = pl.ds(fgn + l0 * PART + off, sz)
            ry = rdma(rows, y_send, y_recv, k, y_peer)
            ry.start()
            ys.append(ry)
            rz = rdma(rows, z_send, z_recv, k, z_peer)
            rz.start()
            zs.append(rz)
            off += sz

        zs[2].wait_recv()
        ry = rdma(
            pl.ds(fgn + l2 * PART + B_OFF, B_LEN), y_send, y_recv, NCH, y_peer
        )
        ry.start()
        ys.append(ry)
        ys[3].wait_recv()
        rz = rdma(
            pl.ds(fgn + l1 * PART + C_OFF, C_LEN), z_send, z_recv, NCH, z_peer
        )
        rz.start()
        zs.append(rz)

        xs[NCH].wait_recv()
        for k in (0, 1, 2, NCH):
            ys[k].wait_recv()
        for k in (0, 1, 3, NCH):
            zs[k].wait_recv()
        for k in range(NSEM):
            xs[k].wait_send()
            ys[k].wait_send()
            zs[k].wait_send()

    return pl.pallas_call(
        body,
        out_shape=jax.ShapeDtypeStruct((2 * m, n), jnp.bfloat16),
        in_specs=[pl.BlockSpec(memory_space=pl.ANY)],
        out_specs=pl.BlockSpec(memory_space=pltpu.VMEM),
        scratch_shapes=[
            pltpu.VMEM((m, n), x.dtype),
            pltpu.SemaphoreType.DMA((NLOAD,)),
            pltpu.SemaphoreType.DMA((NSEM,)),
            pltpu.SemaphoreType.DMA((NSEM,)),
            pltpu.SemaphoreType.DMA((NSEM,)),
            pltpu.SemaphoreType.DMA((NSEM,)),
            pltpu.SemaphoreType.DMA((NSEM,)),
            pltpu.SemaphoreType.DMA((NSEM,)),
        ],
        compiler_params=pltpu.CompilerParams(collective_id=0),
    )(x)
